# Initial kernel scaffold; baseline (speedup 1.0000x reference)
import jax
import jax.numpy as jnp
from jax.experimental import pallas as pl


def kernel(x, edge_attr, l1_src_w, l1_src_b, l1_dst_w, l1_dst_b, l1_edge_w, l1_edge_b, l1_w1, l1_b1, l1_g, l1_be, l1_w2, l1_b2, l2_src_w, l2_src_b, l2_dst_w, l2_dst_b, l2_edge_w, l2_edge_b, l2_w1, l2_b1, l2_g, l2_be, l2_w2, l2_b2, edge_index):
    raise NotImplementedError("write your pallas kernel here")


# speedup vs baseline: 28.8029x; 28.8029x over previous
import functools

import jax
import jax.numpy as jnp
from jax import lax
from jax.experimental import pallas as pl
from jax.experimental.pallas import tpu as pltpu
from jax.experimental.pallas import tpu_sc as plsc

N = 100000
E = 1600000
D = 16
EPS = 1e-7
BN_EPS = 1e-5

_NODE_BLK = 2000
_EDGE_BLK = 4000
_GW = 128
_ROWS_PER_SUBCORE = N // 16
_ZB = 250

_mesh = plsc.VectorSubcoreMesh(core_axis_name="c", subcore_axis_name="s")
_sc_params = pltpu.CompilerParams(use_tc_tiling_on_sc=False)


def _nodeprep_body(x_ref, sw_ref, sb_ref, dw_ref, db_ref, h_ref, hd_ref):
    xb = x_ref[...]
    h_ref[...] = jnp.dot(xb, sw_ref[...], preferred_element_type=jnp.float32) + sb_ref[...]
    hd_ref[...] = jnp.dot(xb, dw_ref[...], preferred_element_type=jnp.float32) + db_ref[...]


def _nodeprep(x, sw, sb, dw, db):
    fin = x.shape[1]
    grid = (N // _NODE_BLK,)
    return pl.pallas_call(
        _nodeprep_body,
        grid=grid,
        in_specs=[
            pl.BlockSpec((_NODE_BLK, fin), lambda i: (i, 0)),
            pl.BlockSpec((fin, D), lambda i: (0, 0)),
            pl.BlockSpec((1, D), lambda i: (0, 0)),
            pl.BlockSpec((fin, D), lambda i: (0, 0)),
            pl.BlockSpec((1, D), lambda i: (0, 0)),
        ],
        out_specs=[
            pl.BlockSpec((_NODE_BLK, D), lambda i: (i, 0)),
            pl.BlockSpec((_NODE_BLK, D), lambda i: (i, 0)),
        ],
        out_shape=[
            jax.ShapeDtypeStruct((N, D), jnp.float32),
            jax.ShapeDtypeStruct((N, D), jnp.float32),
        ],
    )(x, sw, sb.reshape(1, D), dw, db.reshape(1, D))


@functools.partial(
    pl.kernel,
    out_type=jax.ShapeDtypeStruct((E, D), jnp.float32),
    mesh=_mesh,
    compiler_params=_sc_params,
)
def _gather_k(h_hbm, src_hbm, out_hbm):
    def body(i_vmem, o_vmem):
        pltpu.sync_copy(h_hbm.at[i_vmem.at[0]], o_vmem)

    pltpu.emit_pipeline(
        body,
        grid=(E // _GW,),
        in_specs=[pl.BlockSpec((1, _GW), lambda i: (0, i))],
        out_specs=[pl.BlockSpec((_GW, D), lambda i: (i, 0))],
        core_axis_name=("c", "s"),
        dimension_semantics=(pltpu.PARALLEL,),
    )(src_hbm, out_hbm)


def _edge_body(g_ref, ea_ref, ew_ref, eb_ref, w_ref, mw_ref):
    e = jnp.dot(ea_ref[...], ew_ref[...], preferred_element_type=jnp.float32) + eb_ref[...]
    msg = jnp.maximum(g_ref[...] + e, 0.0) + EPS
    w = jnp.exp(msg)
    w_ref[...] = w
    mw_ref[...] = msg * w


def _edge(g, ea, ew, eb):
    grid = (E // _EDGE_BLK,)
    return pl.pallas_call(
        _edge_body,
        grid=grid,
        in_specs=[
            pl.BlockSpec((_EDGE_BLK, D), lambda i: (i, 0)),
            pl.BlockSpec((_EDGE_BLK, 6), lambda i: (i, 0)),
            pl.BlockSpec((6, D), lambda i: (0, 0)),
            pl.BlockSpec((1, D), lambda i: (0, 0)),
        ],
        out_specs=[
            pl.BlockSpec((_EDGE_BLK, D), lambda i: (i, 0)),
            pl.BlockSpec((_EDGE_BLK, D), lambda i: (i, 0)),
        ],
        out_shape=[
            jax.ShapeDtypeStruct((E, D), jnp.float32),
            jax.ShapeDtypeStruct((E, D), jnp.float32),
        ],
    )(g, ea, ew, eb.reshape(1, D))


@functools.partial(
    pl.kernel,
    out_type=jax.ShapeDtypeStruct((2, N, D), jnp.float32),
    mesh=_mesh,
    scratch_types=[
        pltpu.VMEM_SHARED((N, D), jnp.float32),
        pltpu.VMEM((_ZB, D), jnp.float32),
    ],
    compiler_params=_sc_params,
)
def _scatter_k(w_hbm, mw_hbm, dst_hbm, out_hbm, acc, zbuf):
    c = lax.axis_index("c")
    s = lax.axis_index("s")

    @pl.loop(0, _ZB)
    def _(j):
        zbuf.at[j][...] = jnp.zeros((D,), jnp.float32)

    base = s * _ROWS_PER_SUBCORE

    @pl.loop(0, _ROWS_PER_SUBCORE // _ZB)
    def _(k):
        pltpu.sync_copy(zbuf, acc.at[pl.ds(base + k * _ZB, _ZB)])

    plsc.subcore_barrier()

    def body(i_vmem, vals_vmem):
        pltpu.sync_copy(vals_vmem, acc.at[i_vmem.at[0]], add=True)

    def run(vals_hbm):
        pltpu.emit_pipeline(
            body,
            grid=(E // _GW,),
            in_specs=[
                pl.BlockSpec((1, _GW), lambda i: (0, i)),
                pl.BlockSpec((_GW, D), lambda i: (i, 0)),
            ],
            out_specs=[],
            core_axis_name="s",
            dimension_semantics=(pltpu.PARALLEL,),
        )(dst_hbm, vals_hbm)

    @pl.when(c == 0)
    def _():
        run(w_hbm)

    @pl.when(c == 1)
    def _():
        run(mw_hbm)

    plsc.subcore_barrier()
    pltpu.sync_copy(
        acc.at[pl.ds(base, _ROWS_PER_SUBCORE)],
        out_hbm.at[c, pl.ds(base, _ROWS_PER_SUBCORE)],
    )


def _combine1_body(s_ref, t_ref, hd_ref, w1_ref, b1_ref, sc1_ref, be1_ref,
                   w2_ref, b2_ref, sw2_ref, sb2_ref, dw2_ref, db2_ref,
                   h2_ref, hd2_ref):
    out = t_ref[...] / (s_ref[...] + 1e-16) + hd_ref[...]
    h1 = jnp.dot(out, w1_ref[...], preferred_element_type=jnp.float32) + b1_ref[...]
    h1 = h1 * sc1_ref[...] + be1_ref[...]
    h1 = jnp.maximum(h1, 0.0)
    y = jnp.dot(h1, w2_ref[...], preferred_element_type=jnp.float32) + b2_ref[...]
    y = jnp.maximum(y, 0.0)
    h2_ref[...] = jnp.dot(y, sw2_ref[...], preferred_element_type=jnp.float32) + sb2_ref[...]
    hd2_ref[...] = jnp.dot(y, dw2_ref[...], preferred_element_type=jnp.float32) + db2_ref[...]


def _combine1(s, t, hd, w1, b1, scale1, be1, w2, b2, sw2p, sb2p, dw2p, db2p):
    grid = (N // _NODE_BLK,)
    blk = lambda r, cdim: pl.BlockSpec((r, cdim), lambda i: (0, 0))
    return pl.pallas_call(
        _combine1_body,
        grid=grid,
        in_specs=[
            pl.BlockSpec((_NODE_BLK, D), lambda i: (i, 0)),
            pl.BlockSpec((_NODE_BLK, D), lambda i: (i, 0)),
            pl.BlockSpec((_NODE_BLK, D), lambda i: (i, 0)),
            blk(D, 2 * D), blk(1, 2 * D), blk(1, 2 * D), blk(1, 2 * D),
            blk(2 * D, D), blk(1, D),
            blk(D, D), blk(1, D), blk(D, D), blk(1, D),
        ],
        out_specs=[
            pl.BlockSpec((_NODE_BLK, D), lambda i: (i, 0)),
            pl.BlockSpec((_NODE_BLK, D), lambda i: (i, 0)),
        ],
        out_shape=[
            jax.ShapeDtypeStruct((N, D), jnp.float32),
            jax.ShapeDtypeStruct((N, D), jnp.float32),
        ],
    )(s, t, hd, w1, b1.reshape(1, 2 * D), scale1.reshape(1, 2 * D),
      be1.reshape(1, 2 * D), w2, b2.reshape(1, D),
      sw2p, sb2p.reshape(1, D), dw2p, db2p.reshape(1, D))


def _combine2_body(s_ref, t_ref, hd_ref, w1_ref, b1_ref, sc_ref, be_ref,
                   w2_ref, b2_ref, o_ref):
    out = t_ref[:, 0:1] / (s_ref[:, 0:1] + 1e-16) + hd_ref[:, 0:1]
    h1 = out * w1_ref[...] + b1_ref[...]
    h1 = h1 * sc_ref[...] + be_ref[...]
    h1 = jnp.maximum(h1, 0.0)
    z = jnp.dot(h1, w2_ref[...], preferred_element_type=jnp.float32) + b2_ref[...]
    o_ref[...] = jax.nn.sigmoid(z)


def _combine2(s, t, hd, w1, b1, scale, be, w2, b2):
    grid = (N // _NODE_BLK,)
    return pl.pallas_call(
        _combine2_body,
        grid=grid,
        in_specs=[
            pl.BlockSpec((_NODE_BLK, D), lambda i: (i, 0)),
            pl.BlockSpec((_NODE_BLK, D), lambda i: (i, 0)),
            pl.BlockSpec((_NODE_BLK, D), lambda i: (i, 0)),
            pl.BlockSpec((1, 2), lambda i: (0, 0)),
            pl.BlockSpec((1, 2), lambda i: (0, 0)),
            pl.BlockSpec((1, 2), lambda i: (0, 0)),
            pl.BlockSpec((1, 2), lambda i: (0, 0)),
            pl.BlockSpec((2, 1), lambda i: (0, 0)),
            pl.BlockSpec((1, 1), lambda i: (0, 0)),
        ],
        out_specs=pl.BlockSpec((_NODE_BLK, 1), lambda i: (i, 0)),
        out_shape=jax.ShapeDtypeStruct((N, 1), jnp.float32),
    )(s, t, hd, w1, b1.reshape(1, 2), scale.reshape(1, 2), be.reshape(1, 2),
      w2, b2.reshape(1, 1))


def _layer_sparse(h, ea, ew, eb, src2d, dst2d):
    g = _gather_k(h, src2d)
    w, mw = _edge(g, ea, ew, eb)
    st = _scatter_k(w, mw, dst2d)
    return st[0], st[1]


def kernel(x, edge_attr, l1_src_w, l1_src_b, l1_dst_w, l1_dst_b, l1_edge_w,
           l1_edge_b, l1_w1, l1_b1, l1_g, l1_be, l1_w2, l1_b2, l2_src_w,
           l2_src_b, l2_dst_w, l2_dst_b, l2_edge_w, l2_edge_b, l2_w1, l2_b1,
           l2_g, l2_be, l2_w2, l2_b2, edge_index):
    src2d = edge_index[0].reshape(1, E)
    dst2d = edge_index[1].reshape(1, E)

    bn1_scale = l1_g / jnp.sqrt(1.0 + BN_EPS)
    bn2_scale = l2_g / jnp.sqrt(1.0 + BN_EPS)

    pad = lambda w: jnp.pad(w, ((0, 0), (0, D - w.shape[1])))
    sw2p, dw2p = pad(l2_src_w), pad(l2_dst_w)
    sb2p = jnp.pad(l2_src_b, (0, D - 1))
    db2p = jnp.pad(l2_dst_b, (0, D - 1))
    ew2p = pad(l2_edge_w)
    eb2p = jnp.pad(l2_edge_b, (0, D - 1))

    h1, hd1 = _nodeprep(x, l1_src_w, l1_src_b, l1_dst_w, l1_dst_b)
    s1, t1 = _layer_sparse(h1, edge_attr, l1_edge_w, l1_edge_b, src2d, dst2d)
    h2, hd2 = _combine1(s1, t1, hd1, l1_w1, l1_b1, bn1_scale, l1_be, l1_w2,
                        l1_b2, sw2p, sb2p, dw2p, db2p)

    s2, t2 = _layer_sparse(h2, edge_attr, ew2p, eb2p, src2d, dst2d)
    return _combine2(s2, t2, hd2, l2_w1, l2_b1, bn2_scale, l2_be, l2_w2, l2_b2)


# speedup vs baseline: 49.1366x; 1.7060x over previous
import functools

import jax
import jax.numpy as jnp
from jax import lax
from jax.experimental import pallas as pl
from jax.experimental.pallas import tpu as pltpu
from jax.experimental.pallas import tpu_sc as plsc

N = 100000
E = 1600000
D = 16
EPS = 1e-7
BN_EPS = 1e-5

_NODE_BLK = 2000
_EDGE_BLK = 4000
_GW = 128
_ROWS_PER_SUBCORE = N // 16
_ZB = 250

_mesh = plsc.VectorSubcoreMesh(core_axis_name="c", subcore_axis_name="s")
_sc_params = pltpu.CompilerParams(use_tc_tiling_on_sc=False)


_N4 = N // 4
_E8 = E // 8
_NODE4_BLK = 5000
_EDGE8_BLK = 2000


def _kron(w, p):
    return jnp.kron(jnp.eye(p, dtype=jnp.float32), w)


def _tile(b, p):
    return jnp.tile(b, p).reshape(1, -1)


def _nodeprep_body(x_ref, sw_ref, sb_ref, dw_ref, db_ref, h_ref, hd_ref):
    xb = x_ref[...]
    h_ref[...] = jnp.dot(xb, sw_ref[...], preferred_element_type=jnp.float32) + sb_ref[...]
    hd_ref[...] = jnp.dot(xb, dw_ref[...], preferred_element_type=jnp.float32) + db_ref[...]


def _nodeprep(x, sw, sb, dw, db):
    fin = x.shape[1]
    x4 = x.reshape(_N4, 4 * fin)
    grid = (_N4 // _NODE4_BLK,)
    h4, hd4 = pl.pallas_call(
        _nodeprep_body,
        grid=grid,
        in_specs=[
            pl.BlockSpec((_NODE4_BLK, 4 * fin), lambda i: (i, 0)),
            pl.BlockSpec((4 * fin, 4 * D), lambda i: (0, 0)),
            pl.BlockSpec((1, 4 * D), lambda i: (0, 0)),
            pl.BlockSpec((4 * fin, 4 * D), lambda i: (0, 0)),
            pl.BlockSpec((1, 4 * D), lambda i: (0, 0)),
        ],
        out_specs=[
            pl.BlockSpec((_NODE4_BLK, 4 * D), lambda i: (i, 0)),
            pl.BlockSpec((_NODE4_BLK, 4 * D), lambda i: (i, 0)),
        ],
        out_shape=[
            jax.ShapeDtypeStruct((_N4, 4 * D), jnp.float32),
            jax.ShapeDtypeStruct((_N4, 4 * D), jnp.float32),
        ],
    )(x4, _kron(sw, 4), _tile(sb, 4), _kron(dw, 4), _tile(db, 4))
    return h4.reshape(N, D), hd4.reshape(N, D)


@functools.partial(
    pl.kernel,
    out_type=jax.ShapeDtypeStruct((E, D), jnp.float32),
    mesh=_mesh,
    compiler_params=_sc_params,
)
def _gather_k(h_hbm, src_hbm, out_hbm):
    def body(i_vmem, o_vmem):
        pltpu.sync_copy(h_hbm.at[i_vmem.at[0]], o_vmem)

    pltpu.emit_pipeline(
        body,
        grid=(E // _GW,),
        in_specs=[pl.BlockSpec((1, _GW), lambda i: (0, i))],
        out_specs=[pl.BlockSpec((_GW, D), lambda i: (i, 0))],
        core_axis_name=("c", "s"),
        dimension_semantics=(pltpu.PARALLEL,),
    )(src_hbm, out_hbm)


def _edge_body(g_ref, ea_ref, ew_ref, eb_ref, w_ref, mw_ref):
    e = jnp.dot(ea_ref[...], ew_ref[...], preferred_element_type=jnp.float32) + eb_ref[...]
    msg = jnp.maximum(g_ref[...] + e, 0.0) + EPS
    w = jnp.exp(msg)
    w_ref[...] = w
    mw_ref[...] = msg * w


def _edge(g, ea8, ew8, eb8):
    g8 = g.reshape(_E8, 8 * D)
    grid = (_E8 // _EDGE8_BLK,)
    w8, mw8 = pl.pallas_call(
        _edge_body,
        grid=grid,
        in_specs=[
            pl.BlockSpec((_EDGE8_BLK, 8 * D), lambda i: (i, 0)),
            pl.BlockSpec((_EDGE8_BLK, 48), lambda i: (i, 0)),
            pl.BlockSpec((48, 8 * D), lambda i: (0, 0)),
            pl.BlockSpec((1, 8 * D), lambda i: (0, 0)),
        ],
        out_specs=[
            pl.BlockSpec((_EDGE8_BLK, 8 * D), lambda i: (i, 0)),
            pl.BlockSpec((_EDGE8_BLK, 8 * D), lambda i: (i, 0)),
        ],
        out_shape=[
            jax.ShapeDtypeStruct((_E8, 8 * D), jnp.float32),
            jax.ShapeDtypeStruct((_E8, 8 * D), jnp.float32),
        ],
    )(g8, ea8, ew8, eb8)
    return w8.reshape(E, D), mw8.reshape(E, D)


@functools.partial(
    pl.kernel,
    out_type=jax.ShapeDtypeStruct((2, N, D), jnp.float32),
    mesh=_mesh,
    scratch_types=[
        pltpu.VMEM_SHARED((N, D), jnp.float32),
        pltpu.VMEM((_ZB, D), jnp.float32),
    ],
    compiler_params=_sc_params,
)
def _scatter_k(w_hbm, mw_hbm, dst_hbm, out_hbm, acc, zbuf):
    c = lax.axis_index("c")
    s = lax.axis_index("s")

    @pl.loop(0, _ZB)
    def _(j):
        zbuf.at[j][...] = jnp.zeros((D,), jnp.float32)

    base = s * _ROWS_PER_SUBCORE

    @pl.loop(0, _ROWS_PER_SUBCORE // _ZB)
    def _(k):
        pltpu.sync_copy(zbuf, acc.at[pl.ds(base + k * _ZB, _ZB)])

    plsc.subcore_barrier()

    def body(i_vmem, vals_vmem):
        pltpu.sync_copy(vals_vmem, acc.at[i_vmem.at[0]], add=True)

    def run(vals_hbm):
        pltpu.emit_pipeline(
            body,
            grid=(E // _GW,),
            in_specs=[
                pl.BlockSpec((1, _GW), lambda i: (0, i)),
                pl.BlockSpec((_GW, D), lambda i: (i, 0)),
            ],
            out_specs=[],
            core_axis_name="s",
            dimension_semantics=(pltpu.PARALLEL,),
        )(dst_hbm, vals_hbm)

    @pl.when(c == 0)
    def _():
        run(w_hbm)

    @pl.when(c == 1)
    def _():
        run(mw_hbm)

    plsc.subcore_barrier()
    pltpu.sync_copy(
        acc.at[pl.ds(base, _ROWS_PER_SUBCORE)],
        out_hbm.at[c, pl.ds(base, _ROWS_PER_SUBCORE)],
    )


def _combine1_body(s_ref, t_ref, hd_ref, w1_ref, b1_ref, sc1_ref, be1_ref,
                   w2_ref, b2_ref, sw2_ref, sb2_ref, dw2_ref, db2_ref,
                   h2_ref, hd2_ref):
    out = t_ref[...] / (s_ref[...] + 1e-16) + hd_ref[...]
    h1 = jnp.dot(out, w1_ref[...], preferred_element_type=jnp.float32) + b1_ref[...]
    h1 = h1 * sc1_ref[...] + be1_ref[...]
    h1 = jnp.maximum(h1, 0.0)
    y = jnp.dot(h1, w2_ref[...], preferred_element_type=jnp.float32) + b2_ref[...]
    y = jnp.maximum(y, 0.0)
    h2_ref[...] = jnp.dot(y, sw2_ref[...], preferred_element_type=jnp.float32) + sb2_ref[...]
    hd2_ref[...] = jnp.dot(y, dw2_ref[...], preferred_element_type=jnp.float32) + db2_ref[...]


def _combine1(s, t, hd, w1, b1, scale1, be1, w2, b2, sw2p, sb2p, dw2p, db2p):
    s4 = s.reshape(_N4, 4 * D)
    t4 = t.reshape(_N4, 4 * D)
    hd4 = hd.reshape(_N4, 4 * D)
    grid = (_N4 // _NODE4_BLK,)
    blk = lambda r, cdim: pl.BlockSpec((r, cdim), lambda i: (0, 0))
    h2, hd2 = pl.pallas_call(
        _combine1_body,
        grid=grid,
        in_specs=[
            pl.BlockSpec((_NODE4_BLK, 4 * D), lambda i: (i, 0)),
            pl.BlockSpec((_NODE4_BLK, 4 * D), lambda i: (i, 0)),
            pl.BlockSpec((_NODE4_BLK, 4 * D), lambda i: (i, 0)),
            blk(4 * D, 8 * D), blk(1, 8 * D), blk(1, 8 * D), blk(1, 8 * D),
            blk(8 * D, 4 * D), blk(1, 4 * D),
            blk(4 * D, 4 * D), blk(1, 4 * D), blk(4 * D, 4 * D), blk(1, 4 * D),
        ],
        out_specs=[
            pl.BlockSpec((_NODE4_BLK, 4 * D), lambda i: (i, 0)),
            pl.BlockSpec((_NODE4_BLK, 4 * D), lambda i: (i, 0)),
        ],
        out_shape=[
            jax.ShapeDtypeStruct((_N4, 4 * D), jnp.float32),
            jax.ShapeDtypeStruct((_N4, 4 * D), jnp.float32),
        ],
    )(s4, t4, hd4, _kron(w1, 4), _tile(b1, 4), _tile(scale1, 4),
      _tile(be1, 4), _kron(w2, 4), _tile(b2, 4),
      _kron(sw2p, 4), _tile(sb2p, 4), _kron(dw2p, 4), _tile(db2p, 4))
    return h2.reshape(N, D), hd2.reshape(N, D)


def _combine2_body(s_ref, t_ref, hd_ref, w1_ref, b1_ref, sc_ref, be_ref,
                   w2_ref, b2_ref, o_ref):
    out = t_ref[:, 0:1] / (s_ref[:, 0:1] + 1e-16) + hd_ref[:, 0:1]
    h1 = out * w1_ref[...] + b1_ref[...]
    h1 = h1 * sc_ref[...] + be_ref[...]
    h1 = jnp.maximum(h1, 0.0)
    z = jnp.dot(h1, w2_ref[...], preferred_element_type=jnp.float32) + b2_ref[...]
    o_ref[...] = jax.nn.sigmoid(z)


def _combine2(s, t, hd, w1, b1, scale, be, w2, b2):
    grid = (N // _NODE_BLK,)
    return pl.pallas_call(
        _combine2_body,
        grid=grid,
        in_specs=[
            pl.BlockSpec((_NODE_BLK, D), lambda i: (i, 0)),
            pl.BlockSpec((_NODE_BLK, D), lambda i: (i, 0)),
            pl.BlockSpec((_NODE_BLK, D), lambda i: (i, 0)),
            pl.BlockSpec((1, 2), lambda i: (0, 0)),
            pl.BlockSpec((1, 2), lambda i: (0, 0)),
            pl.BlockSpec((1, 2), lambda i: (0, 0)),
            pl.BlockSpec((1, 2), lambda i: (0, 0)),
            pl.BlockSpec((2, 1), lambda i: (0, 0)),
            pl.BlockSpec((1, 1), lambda i: (0, 0)),
        ],
        out_specs=pl.BlockSpec((_NODE_BLK, 1), lambda i: (i, 0)),
        out_shape=jax.ShapeDtypeStruct((N, 1), jnp.float32),
    )(s, t, hd, w1, b1.reshape(1, 2), scale.reshape(1, 2), be.reshape(1, 2),
      w2, b2.reshape(1, 1))


def _layer_sparse(h, ea8, ew8, eb8, src2d, dst2d):
    g = _gather_k(h, src2d)
    w, mw = _edge(g, ea8, ew8, eb8)
    st = _scatter_k(w, mw, dst2d)
    return st[0], st[1]


def kernel(x, edge_attr, l1_src_w, l1_src_b, l1_dst_w, l1_dst_b, l1_edge_w,
           l1_edge_b, l1_w1, l1_b1, l1_g, l1_be, l1_w2, l1_b2, l2_src_w,
           l2_src_b, l2_dst_w, l2_dst_b, l2_edge_w, l2_edge_b, l2_w1, l2_b1,
           l2_g, l2_be, l2_w2, l2_b2, edge_index):
    src2d = edge_index[0].reshape(1, E)
    dst2d = edge_index[1].reshape(1, E)

    bn1_scale = l1_g / jnp.sqrt(1.0 + BN_EPS)
    bn2_scale = l2_g / jnp.sqrt(1.0 + BN_EPS)

    pad = lambda w: jnp.pad(w, ((0, 0), (0, D - w.shape[1])))
    sw2p, dw2p = pad(l2_src_w), pad(l2_dst_w)
    sb2p = jnp.pad(l2_src_b, (0, D - 1))
    db2p = jnp.pad(l2_dst_b, (0, D - 1))
    ew2p = pad(l2_edge_w)
    eb2p = jnp.pad(l2_edge_b, (0, D - 1))

    ea8 = edge_attr.reshape(_E8, 48)

    h1, hd1 = _nodeprep(x, l1_src_w, l1_src_b, l1_dst_w, l1_dst_b)
    s1, t1 = _layer_sparse(h1, ea8, _kron(l1_edge_w, 8), _tile(l1_edge_b, 8),
                           src2d, dst2d)
    h2, hd2 = _combine1(s1, t1, hd1, l1_w1, l1_b1, bn1_scale, l1_be, l1_w2,
                        l1_b2, sw2p, sb2p, dw2p, db2p)

    s2, t2 = _layer_sparse(h2, ea8, _kron(ew2p, 8), _tile(eb2p, 8), src2d, dst2d)
    return _combine2(s2, t2, hd2, l2_w1, l2_b1, bn2_scale, l2_be, l2_w2, l2_b2)
